# baseline (speedup 1.0000x reference)
import functools

import jax
import jax.numpy as jnp
from jax import lax
from jax.experimental import pallas as pl
from jax.experimental.pallas import tpu as pltpu
from jax.experimental.pallas import tpu_sc as plsc

N_NODES = 10000
D = 128

NUM_SC = 2
NUM_TILES = 16
NUM_WORKERS = NUM_SC * NUM_TILES
LANES = 16

CHUNK = 128
CHUNKS_C0 = 48
CHUNKS_C1 = 112
CHUNKS_PER_PAIR = CHUNKS_C0 + CHUNKS_C1
E_PAD = CHUNK * CHUNKS_PER_PAIR * NUM_TILES
N_PAD = 10240
ROWS_PER_TILE = N_PAD // NUM_TILES


NBUF = 2


def _sc_segment_sum(ego, packed2, w2, zeros_tile):
    mesh = plsc.VectorSubcoreMesh(
        core_axis_name="c", subcore_axis_name="s",
        num_cores=NUM_SC, num_subcores=NUM_TILES)

    rows_types = [pltpu.VMEM((CHUNK, D), jnp.float32) for _ in range(NBUF)]
    idx_types = [pltpu.VMEM((CHUNK,), jnp.int32) for _ in range(2 * NBUF)]
    wbuf_types = [pltpu.VMEM((CHUNK,), jnp.float32) for _ in range(NBUF)]
    sem_types = [pltpu.SemaphoreType.DMA for _ in range(2 * NBUF)]

    @functools.partial(
        pl.kernel,
        out_type=jax.ShapeDtypeStruct((NUM_SC, N_PAD, D), jnp.float32),
        mesh=mesh,
        scratch_types=[
            pltpu.VMEM((CHUNKS_C1, CHUNK), jnp.int32),
            pltpu.VMEM_SHARED((N_PAD, D), jnp.float32),
        ] + rows_types + idx_types + wbuf_types + sem_types,
    )
    def k(ego_hbm, packed_hbm, w_hbm, zeros_hbm, out_hbm,
          packed_all, acc, *rest):
        rows = rest[:NBUF]
        src_v = rest[NBUF:2 * NBUF]
        dst_v = rest[2 * NBUF:3 * NBUF]
        wbuf = rest[3 * NBUF:4 * NBUF]
        gsems = rest[4 * NBUF:5 * NBUF]
        wsems = rest[5 * NBUF:6 * NBUF]
        cid = lax.axis_index("c")
        sid = lax.axis_index("s")
        cbase = sid * CHUNKS_PER_PAIR + jnp.where(cid == 0, 0, CHUNKS_C0)
        nouter = jnp.where(cid == 0, CHUNKS_C0 // NBUF, CHUNKS_C1 // NBUF)

        pltpu.sync_copy(packed_hbm.at[pl.ds(cbase, CHUNKS_C1)], packed_all)

        pltpu.sync_copy(zeros_hbm, acc.at[pl.ds(sid * ROWS_PER_TILE, ROWS_PER_TILE)])

        def unpack(t, out_ref, shift, mask):
            for g in range(CHUNK // LANES):
                sl = pl.ds(g * LANES, LANES)
                v = packed_all[t, sl]
                out_ref[sl] = (v >> shift) & mask

        def start_gather(t, b):
            unpack(t, src_v[b], 0, 0xFFFF)
            pltpu.async_copy(ego_hbm.at[src_v[b]], rows[b], gsems[b])
            pltpu.async_copy(w_hbm.at[cbase + t], wbuf[b], wsems[b])

        plsc.subcore_barrier()

        for b in range(NBUF):
            start_gather(b, b)

        def scale(b):
            def group_body(g, _):
                wvec = wbuf[b][pl.ds(g * LANES, LANES)]
                for i in range(LANES):
                    wv = jnp.full((LANES,), wvec[i], jnp.float32)
                    j = g * LANES + i
                    for q in range(D // LANES):
                        sl = pl.ds(q * LANES, LANES)
                        rows[b][j, sl] = rows[b][j, sl] * wv
                return 0

            lax.fori_loop(0, CHUNK // LANES, group_body, 0)

        def outer(g, _):
            for b in range(NBUF):
                t = g * NBUF + b
                pltpu.make_async_copy(
                    ego_hbm.at[src_v[b]], rows[b], gsems[b]).wait()
                pltpu.make_async_copy(
                    w_hbm.at[cbase + t], wbuf[b], wsems[b]).wait()
                scale(b)
                unpack(t, dst_v[b], 16, 0x3FFF)
                pltpu.sync_copy(rows[b], acc.at[dst_v[b]], add=True)

                @pl.when(g < nouter - 1)
                def _():
                    start_gather(t + NBUF, b)
            return 0

        lax.fori_loop(0, nouter, outer, 0)
        plsc.subcore_barrier()

        sl = pl.ds(sid * ROWS_PER_TILE, ROWS_PER_TILE)
        pltpu.sync_copy(acc.at[sl], out_hbm.at[cid].at[sl])

    return k(ego, packed2, w2, zeros_tile)


def _dense_body(ego_ref, lp_ref, w1_ref, w2_ref, b1_ref, b2_ref, out_ref):
    ego = ego_ref[...]
    lap = lp_ref[0] + lp_ref[1]
    y = jnp.dot(ego + lap, w1_ref[...], preferred_element_type=jnp.float32)
    y += jnp.dot(ego * lap, w2_ref[...], preferred_element_type=jnp.float32)
    y += b1_ref[...] + b2_ref[...]
    y = jnp.where(y >= 0, y, 0.2 * y)
    norm = jnp.sqrt(jnp.sum(y * y, axis=1, keepdims=True))
    out_ref[...] = y / jnp.maximum(norm, 1e-12)


def _dense(ego, partials, W1, W2, b1, b2):
    R = 2000
    grid = (N_NODES // R,)
    row_spec = pl.BlockSpec((R, D), lambda i: (i, 0))
    part_spec = pl.BlockSpec((NUM_SC, R, D), lambda i: (0, i, 0))
    full_spec = pl.BlockSpec((D, D), lambda i: (0, 0))
    bias_spec = pl.BlockSpec((1, D), lambda i: (0, 0))
    return pl.pallas_call(
        _dense_body,
        grid=grid,
        in_specs=[row_spec, part_spec, full_spec, full_spec,
                  bias_spec, bias_spec],
        out_specs=row_spec,
        out_shape=jax.ShapeDtypeStruct((N_NODES, D), jnp.float32),
    )(ego, partials, W1, W2, b1, b2)


@jax.jit
def kernel(ego_embeddings, edge_index, edge_weight, W1, W2, b1, b2):
    e = edge_index.shape[1]
    src = edge_index[0].astype(jnp.int32)
    dst = edge_index[1].astype(jnp.int32)
    w = edge_weight.astype(jnp.float32)
    pad = E_PAD - e
    nchunks = NUM_TILES * CHUNKS_PER_PAIR
    packed = src | (dst << 16)
    packed = jnp.concatenate([packed, jnp.zeros((pad,), jnp.int32)])
    packed = packed.reshape(nchunks, CHUNK)
    w = jnp.concatenate([w, jnp.zeros((pad,), jnp.float32)]).reshape(nchunks, CHUNK)
    zeros_tile = jnp.zeros((ROWS_PER_TILE, D), jnp.float32)

    partials = _sc_segment_sum(ego_embeddings, packed, w, zeros_tile)
    return _dense(ego_embeddings, partials, W1, W2, b1, b2)


# speedup vs baseline: 1.0716x; 1.0716x over previous
import functools

import jax
import jax.numpy as jnp
from jax import lax
from jax.experimental import pallas as pl
from jax.experimental.pallas import tpu as pltpu
from jax.experimental.pallas import tpu_sc as plsc

N_NODES = 10000
D = 128

NUM_SC = 2
NUM_TILES = 16
NUM_WORKERS = NUM_SC * NUM_TILES
LANES = 16

CHUNK = 128
CHUNKS_C0 = 80
CHUNKS_C1 = 80
CHUNKS_PER_PAIR = CHUNKS_C0 + CHUNKS_C1
E_PAD = CHUNK * CHUNKS_PER_PAIR * NUM_TILES
N_PAD = 10240
ROWS_PER_TILE = N_PAD // NUM_TILES


NBUF = 2


def _sc_segment_sum(ego, packed2, w2, zeros_tile):
    mesh = plsc.VectorSubcoreMesh(
        core_axis_name="c", subcore_axis_name="s",
        num_cores=NUM_SC, num_subcores=NUM_TILES)

    rows_types = [pltpu.VMEM((CHUNK, D), jnp.float32) for _ in range(NBUF)]
    idx_types = [pltpu.VMEM((CHUNK,), jnp.int32) for _ in range(2 * NBUF)]
    wbuf_types = [pltpu.VMEM((CHUNK,), jnp.float32) for _ in range(NBUF)]
    sem_types = [pltpu.SemaphoreType.DMA for _ in range(2 * NBUF)]

    @functools.partial(
        pl.kernel,
        out_type=jax.ShapeDtypeStruct((NUM_SC, N_PAD, D), jnp.float32),
        mesh=mesh,
        scratch_types=[
            pltpu.VMEM((CHUNKS_C1, CHUNK), jnp.int32),
            pltpu.VMEM_SHARED((N_PAD, D), jnp.float32),
        ] + rows_types + idx_types + wbuf_types + sem_types,
    )
    def k(ego_hbm, packed_hbm, w_hbm, zeros_hbm, out_hbm,
          packed_all, acc, *rest):
        rows = rest[:NBUF]
        src_v = rest[NBUF:2 * NBUF]
        dst_v = rest[2 * NBUF:3 * NBUF]
        wbuf = rest[3 * NBUF:4 * NBUF]
        gsems = rest[4 * NBUF:5 * NBUF]
        wsems = rest[5 * NBUF:6 * NBUF]
        cid = lax.axis_index("c")
        sid = lax.axis_index("s")
        cbase = sid * CHUNKS_PER_PAIR + jnp.where(cid == 0, 0, CHUNKS_C0)
        nouter = jnp.where(cid == 0, CHUNKS_C0 // NBUF, CHUNKS_C1 // NBUF)

        pltpu.sync_copy(packed_hbm.at[pl.ds(cbase, CHUNKS_C1)], packed_all)

        pltpu.sync_copy(zeros_hbm, acc.at[pl.ds(sid * ROWS_PER_TILE, ROWS_PER_TILE)])

        def unpack(t, out_ref, shift, mask):
            for g in range(CHUNK // LANES):
                sl = pl.ds(g * LANES, LANES)
                v = packed_all[t, sl]
                out_ref[sl] = (v >> shift) & mask

        def start_gather(t, b):
            unpack(t, src_v[b], 0, 0xFFFF)
            pltpu.async_copy(ego_hbm.at[src_v[b]], rows[b], gsems[b])
            pltpu.async_copy(w_hbm.at[cbase + t], wbuf[b], wsems[b])

        plsc.subcore_barrier()

        for b in range(NBUF):
            start_gather(b, b)

        def scale(b):
            def group_body(g, _):
                wvec = wbuf[b][pl.ds(g * LANES, LANES)]
                for i in range(LANES):
                    wv = jnp.full((LANES,), wvec[i], jnp.float32)
                    j = g * LANES + i
                    for q in range(D // LANES):
                        sl = pl.ds(q * LANES, LANES)
                        rows[b][j, sl] = rows[b][j, sl] * wv
                return 0

            lax.fori_loop(0, CHUNK // LANES, group_body, 0)

        def outer(g, _):
            for b in range(NBUF):
                t = g * NBUF + b
                pltpu.make_async_copy(
                    ego_hbm.at[src_v[b]], rows[b], gsems[b]).wait()
                pltpu.make_async_copy(
                    w_hbm.at[cbase + t], wbuf[b], wsems[b]).wait()
                scale(b)
                unpack(t, dst_v[b], 16, 0x3FFF)
                pltpu.sync_copy(rows[b], acc.at[dst_v[b]], add=True)

                @pl.when(g < nouter - 1)
                def _():
                    start_gather(t + NBUF, b)
            return 0

        lax.fori_loop(0, nouter, outer, 0)
        plsc.subcore_barrier()

        sl = pl.ds(sid * ROWS_PER_TILE, ROWS_PER_TILE)
        pltpu.sync_copy(acc.at[sl], out_hbm.at[cid].at[sl])

    return k(ego, packed2, w2, zeros_tile)


_E_ROWS = 2500


def _prep_body(ei_ref, w_ref, packed_ref, wout_ref):
    src = ei_ref[0]
    dst = ei_ref[1]
    packed_ref[:_E_ROWS] = src | (dst << 16)
    packed_ref[_E_ROWS:] = jnp.zeros((packed_ref.shape[0] - _E_ROWS, D), jnp.int32)
    wout_ref[:_E_ROWS] = w_ref[...]
    wout_ref[_E_ROWS:] = jnp.zeros((wout_ref.shape[0] - _E_ROWS, D), jnp.float32)


def _prep(edge_index, edge_weight):
    nchunks = NUM_TILES * CHUNKS_PER_PAIR
    return pl.pallas_call(
        _prep_body,
        out_shape=(jax.ShapeDtypeStruct((nchunks, CHUNK), jnp.int32),
                   jax.ShapeDtypeStruct((nchunks, CHUNK), jnp.float32)),
    )(edge_index, edge_weight)


def _dense_body(ego_ref, lp_ref, w1_ref, w2_ref, b1_ref, b2_ref, out_ref):
    ego = ego_ref[...]
    lap = lp_ref[0] + lp_ref[1]
    y = jnp.dot(ego + lap, w1_ref[...], preferred_element_type=jnp.float32)
    y += jnp.dot(ego * lap, w2_ref[...], preferred_element_type=jnp.float32)
    y += b1_ref[...] + b2_ref[...]
    y = jnp.where(y >= 0, y, 0.2 * y)
    norm = jnp.sqrt(jnp.sum(y * y, axis=1, keepdims=True))
    out_ref[...] = y / jnp.maximum(norm, 1e-12)


def _dense(ego, partials, W1, W2, b1, b2):
    R = 2000
    grid = (N_NODES // R,)
    row_spec = pl.BlockSpec((R, D), lambda i: (i, 0))
    part_spec = pl.BlockSpec((NUM_SC, R, D), lambda i: (0, i, 0))
    full_spec = pl.BlockSpec((D, D), lambda i: (0, 0))
    bias_spec = pl.BlockSpec((1, D), lambda i: (0, 0))
    return pl.pallas_call(
        _dense_body,
        grid=grid,
        in_specs=[row_spec, part_spec, full_spec, full_spec,
                  bias_spec, bias_spec],
        out_specs=row_spec,
        out_shape=jax.ShapeDtypeStruct((N_NODES, D), jnp.float32),
    )(ego, partials, W1, W2, b1, b2)


@jax.jit
def kernel(ego_embeddings, edge_index, edge_weight, W1, W2, b1, b2):
    ei = edge_index.astype(jnp.int32).reshape(2, _E_ROWS, CHUNK)
    wr = edge_weight.astype(jnp.float32).reshape(_E_ROWS, CHUNK)
    packed, w = _prep(ei, wr)
    zeros_tile = jnp.zeros((ROWS_PER_TILE, D), jnp.float32)

    partials = _sc_segment_sum(ego_embeddings, packed, w, zeros_tile)
    return _dense(ego_embeddings, partials, W1, W2, b1, b2)


# speedup vs baseline: 1.0841x; 1.0117x over previous
import functools

import jax
import jax.numpy as jnp
from jax import lax
from jax.experimental import pallas as pl
from jax.experimental.pallas import tpu as pltpu
from jax.experimental.pallas import tpu_sc as plsc

N_NODES = 10000
D = 128

NUM_SC = 2
NUM_TILES = 16
NUM_WORKERS = NUM_SC * NUM_TILES
LANES = 16

CHUNK = 128
CHUNKS_C0 = 80
CHUNKS_C1 = 80
CHUNKS_PER_PAIR = CHUNKS_C0 + CHUNKS_C1
E_PAD = CHUNK * CHUNKS_PER_PAIR * NUM_TILES
N_PAD = 10240
ROWS_PER_TILE = N_PAD // NUM_TILES


NBUF = 2


def _sc_segment_sum(ego, packed2, w2):
    mesh = plsc.VectorSubcoreMesh(
        core_axis_name="c", subcore_axis_name="s",
        num_cores=NUM_SC, num_subcores=NUM_TILES)

    rows_types = [pltpu.VMEM((CHUNK, D), jnp.float32) for _ in range(NBUF)]
    idx_types = [pltpu.VMEM((CHUNK,), jnp.int32) for _ in range(2 * NBUF)]
    wbuf_types = [pltpu.VMEM((CHUNK,), jnp.float32) for _ in range(NBUF)]
    sem_types = [pltpu.SemaphoreType.DMA for _ in range(2 * NBUF)]

    @functools.partial(
        pl.kernel,
        out_type=jax.ShapeDtypeStruct((NUM_SC, N_PAD, D), jnp.float32),
        mesh=mesh,
        scratch_types=[
            pltpu.VMEM((CHUNKS_C1, CHUNK), jnp.int32),
            pltpu.VMEM_SHARED((N_PAD, D), jnp.float32),
        ] + rows_types + idx_types + wbuf_types + sem_types,
    )
    def k(ego_hbm, packed_hbm, w_hbm, out_hbm, packed_all, acc, *rest):
        rows = rest[:NBUF]
        src_v = rest[NBUF:2 * NBUF]
        dst_v = rest[2 * NBUF:3 * NBUF]
        wbuf = rest[3 * NBUF:4 * NBUF]
        gsems = rest[4 * NBUF:5 * NBUF]
        wsems = rest[5 * NBUF:6 * NBUF]
        cid = lax.axis_index("c")
        sid = lax.axis_index("s")
        cbase = sid * CHUNKS_PER_PAIR + jnp.where(cid == 0, 0, CHUNKS_C0)
        nouter = jnp.where(cid == 0, CHUNKS_C0 // NBUF, CHUNKS_C1 // NBUF)

        pltpu.sync_copy(packed_hbm.at[pl.ds(cbase, CHUNKS_C1)], packed_all)

        def zrow(r, _):
            for q in range(D // LANES):
                rows[0][r, pl.ds(q * LANES, LANES)] = jnp.zeros((LANES,), jnp.float32)
            return 0

        lax.fori_loop(0, CHUNK, zrow, 0)
        for r in range(ROWS_PER_TILE // CHUNK):
            pltpu.sync_copy(
                rows[0], acc.at[pl.ds(sid * ROWS_PER_TILE + r * CHUNK, CHUNK)])

        def unpack(t, out_ref, shift, mask):
            for g in range(CHUNK // LANES):
                sl = pl.ds(g * LANES, LANES)
                v = packed_all[t, sl]
                out_ref[sl] = (v >> shift) & mask

        def start_gather(t, b):
            unpack(t, src_v[b], 0, 0xFFFF)
            pltpu.async_copy(ego_hbm.at[src_v[b]], rows[b], gsems[b])
            pltpu.async_copy(w_hbm.at[cbase + t], wbuf[b], wsems[b])

        plsc.subcore_barrier()

        for b in range(NBUF):
            start_gather(b, b)

        def scale(b):
            def group_body(g, _):
                wvec = wbuf[b][pl.ds(g * LANES, LANES)]
                for i in range(LANES):
                    wv = jnp.full((LANES,), wvec[i], jnp.float32)
                    j = g * LANES + i
                    for q in range(D // LANES):
                        sl = pl.ds(q * LANES, LANES)
                        rows[b][j, sl] = rows[b][j, sl] * wv
                return 0

            lax.fori_loop(0, CHUNK // LANES, group_body, 0)

        def outer(g, _):
            for b in range(NBUF):
                t = g * NBUF + b
                pltpu.make_async_copy(
                    ego_hbm.at[src_v[b]], rows[b], gsems[b]).wait()
                pltpu.make_async_copy(
                    w_hbm.at[cbase + t], wbuf[b], wsems[b]).wait()
                scale(b)
                unpack(t, dst_v[b], 16, 0x3FFF)
                pltpu.sync_copy(rows[b], acc.at[dst_v[b]], add=True)

                @pl.when(g < nouter - 1)
                def _():
                    start_gather(t + NBUF, b)
            return 0

        lax.fori_loop(0, nouter, outer, 0)
        plsc.subcore_barrier()

        sl = pl.ds(sid * ROWS_PER_TILE, ROWS_PER_TILE)
        pltpu.sync_copy(acc.at[sl], out_hbm.at[cid].at[sl])

    return k(ego, packed2, w2)


_E_ROWS = 2500


def _prep_body(ei_ref, w_ref, packed_ref, wout_ref):
    src = ei_ref[0]
    dst = ei_ref[1]
    packed_ref[:_E_ROWS] = src | (dst << 16)
    packed_ref[_E_ROWS:] = jnp.zeros((packed_ref.shape[0] - _E_ROWS, D), jnp.int32)
    wout_ref[:_E_ROWS] = w_ref[...]
    wout_ref[_E_ROWS:] = jnp.zeros((wout_ref.shape[0] - _E_ROWS, D), jnp.float32)


def _prep(edge_index, edge_weight):
    nchunks = NUM_TILES * CHUNKS_PER_PAIR
    return pl.pallas_call(
        _prep_body,
        out_shape=(jax.ShapeDtypeStruct((nchunks, CHUNK), jnp.int32),
                   jax.ShapeDtypeStruct((nchunks, CHUNK), jnp.float32)),
    )(edge_index, edge_weight)


def _dense_body(ego_ref, lp_ref, w1_ref, w2_ref, b1_ref, b2_ref, out_ref):
    ego = ego_ref[...]
    lap = lp_ref[0] + lp_ref[1]
    y = jnp.dot(ego + lap, w1_ref[...], preferred_element_type=jnp.float32)
    y += jnp.dot(ego * lap, w2_ref[...], preferred_element_type=jnp.float32)
    y += b1_ref[...] + b2_ref[...]
    y = jnp.where(y >= 0, y, 0.2 * y)
    norm = jnp.sqrt(jnp.sum(y * y, axis=1, keepdims=True))
    out_ref[...] = y / jnp.maximum(norm, 1e-12)


def _dense(ego, partials, W1, W2, b1, b2):
    R = 2000
    grid = (N_NODES // R,)
    row_spec = pl.BlockSpec((R, D), lambda i: (i, 0))
    part_spec = pl.BlockSpec((NUM_SC, R, D), lambda i: (0, i, 0))
    full_spec = pl.BlockSpec((D, D), lambda i: (0, 0))
    bias_spec = pl.BlockSpec((1, D), lambda i: (0, 0))
    return pl.pallas_call(
        _dense_body,
        grid=grid,
        in_specs=[row_spec, part_spec, full_spec, full_spec,
                  bias_spec, bias_spec],
        out_specs=row_spec,
        out_shape=jax.ShapeDtypeStruct((N_NODES, D), jnp.float32),
    )(ego, partials, W1, W2, b1, b2)


@jax.jit
def kernel(ego_embeddings, edge_index, edge_weight, W1, W2, b1, b2):
    ei = edge_index.astype(jnp.int32).reshape(2, _E_ROWS, CHUNK)
    wr = edge_weight.astype(jnp.float32).reshape(_E_ROWS, CHUNK)
    packed, w = _prep(ei, wr)
    partials = _sc_segment_sum(ego_embeddings, packed, w)
    return _dense(ego_embeddings, partials, W1, W2, b1, b2)


# speedup vs baseline: 1.1131x; 1.0267x over previous
import functools

import jax
import jax.numpy as jnp
from jax import lax
from jax.experimental import pallas as pl
from jax.experimental.pallas import tpu as pltpu
from jax.experimental.pallas import tpu_sc as plsc

N_NODES = 10000
D = 128

NUM_SC = 2
NUM_TILES = 16
NUM_WORKERS = NUM_SC * NUM_TILES
LANES = 16

CHUNK = 128
CHUNKS_C0 = 112
CHUNKS_C1 = 48
CHUNKS_PER_PAIR = CHUNKS_C0 + CHUNKS_C1
E_PAD = CHUNK * CHUNKS_PER_PAIR * NUM_TILES
N_PAD = 10240
ROWS_PER_TILE = N_PAD // NUM_TILES


NBUF = 2


def _sc_segment_sum(ego, packed2, w2):
    mesh = plsc.VectorSubcoreMesh(
        core_axis_name="c", subcore_axis_name="s",
        num_cores=NUM_SC, num_subcores=NUM_TILES)

    rows_types = [pltpu.VMEM((CHUNK, D), jnp.float32) for _ in range(NBUF)]
    idx_types = [pltpu.VMEM((CHUNK,), jnp.int32) for _ in range(2 * NBUF)]
    wbuf_types = [pltpu.VMEM((CHUNK,), jnp.float32) for _ in range(NBUF)]
    sem_types = [pltpu.SemaphoreType.DMA for _ in range(2 * NBUF)]

    @functools.partial(
        pl.kernel,
        out_type=jax.ShapeDtypeStruct((NUM_SC, N_PAD, D), jnp.float32),
        mesh=mesh,
        scratch_types=[
            pltpu.VMEM((max(CHUNKS_C0, CHUNKS_C1), CHUNK), jnp.int32),
            pltpu.VMEM_SHARED((N_PAD, D), jnp.float32),
        ] + rows_types + idx_types + wbuf_types + sem_types,
    )
    def k(ego_hbm, packed_hbm, w_hbm, out_hbm, packed_all, acc, *rest):
        rows = rest[:NBUF]
        src_v = rest[NBUF:2 * NBUF]
        dst_v = rest[2 * NBUF:3 * NBUF]
        wbuf = rest[3 * NBUF:4 * NBUF]
        gsems = rest[4 * NBUF:5 * NBUF]
        wsems = rest[5 * NBUF:6 * NBUF]
        cid = lax.axis_index("c")
        sid = lax.axis_index("s")
        cbase = sid * CHUNKS_PER_PAIR + jnp.where(cid == 0, 0, CHUNKS_C0)
        nouter = jnp.where(cid == 0, CHUNKS_C0 // NBUF, CHUNKS_C1 // NBUF)

        @pl.when(cid == 0)
        def _():
            pltpu.sync_copy(packed_hbm.at[pl.ds(cbase, CHUNKS_C0)],
                            packed_all.at[pl.ds(0, CHUNKS_C0)])

        @pl.when(cid != 0)
        def _():
            pltpu.sync_copy(packed_hbm.at[pl.ds(cbase, CHUNKS_C1)],
                            packed_all.at[pl.ds(0, CHUNKS_C1)])

        def zrow(r, _):
            for q in range(D // LANES):
                rows[0][r, pl.ds(q * LANES, LANES)] = jnp.zeros((LANES,), jnp.float32)
            return 0

        lax.fori_loop(0, CHUNK, zrow, 0)
        for r in range(ROWS_PER_TILE // CHUNK):
            pltpu.sync_copy(
                rows[0], acc.at[pl.ds(sid * ROWS_PER_TILE + r * CHUNK, CHUNK)])

        def unpack(t, out_ref, shift, mask):
            for g in range(CHUNK // LANES):
                sl = pl.ds(g * LANES, LANES)
                v = packed_all[t, sl]
                out_ref[sl] = (v >> shift) & mask

        def start_gather(t, b):
            unpack(t, src_v[b], 0, 0xFFFF)
            pltpu.async_copy(ego_hbm.at[src_v[b]], rows[b], gsems[b])
            pltpu.async_copy(w_hbm.at[cbase + t], wbuf[b], wsems[b])

        plsc.subcore_barrier()

        for b in range(NBUF):
            start_gather(b, b)

        def scale(b):
            def group_body(g, _):
                wvec = wbuf[b][pl.ds(g * LANES, LANES)]
                for i in range(LANES):
                    wv = jnp.full((LANES,), wvec[i], jnp.float32)
                    j = g * LANES + i
                    for q in range(D // LANES):
                        sl = pl.ds(q * LANES, LANES)
                        rows[b][j, sl] = rows[b][j, sl] * wv
                return 0

            lax.fori_loop(0, CHUNK // LANES, group_body, 0)

        def outer(g, _):
            for b in range(NBUF):
                t = g * NBUF + b
                pltpu.make_async_copy(
                    ego_hbm.at[src_v[b]], rows[b], gsems[b]).wait()
                pltpu.make_async_copy(
                    w_hbm.at[cbase + t], wbuf[b], wsems[b]).wait()
                scale(b)
                unpack(t, dst_v[b], 16, 0x3FFF)
                pltpu.sync_copy(rows[b], acc.at[dst_v[b]], add=True)

                @pl.when(g < nouter - 1)
                def _():
                    start_gather(t + NBUF, b)
            return 0

        lax.fori_loop(0, nouter, outer, 0)
        plsc.subcore_barrier()

        sl = pl.ds(sid * ROWS_PER_TILE, ROWS_PER_TILE)
        pltpu.sync_copy(acc.at[sl], out_hbm.at[cid].at[sl])

    return k(ego, packed2, w2)


_E_ROWS = 2500


def _prep_body(ei_ref, w_ref, packed_ref, wout_ref):
    src = ei_ref[0]
    dst = ei_ref[1]
    packed_ref[:_E_ROWS] = src | (dst << 16)
    packed_ref[_E_ROWS:] = jnp.zeros((packed_ref.shape[0] - _E_ROWS, D), jnp.int32)
    wout_ref[:_E_ROWS] = w_ref[...]
    wout_ref[_E_ROWS:] = jnp.zeros((wout_ref.shape[0] - _E_ROWS, D), jnp.float32)


def _prep(edge_index, edge_weight):
    nchunks = NUM_TILES * CHUNKS_PER_PAIR
    return pl.pallas_call(
        _prep_body,
        out_shape=(jax.ShapeDtypeStruct((nchunks, CHUNK), jnp.int32),
                   jax.ShapeDtypeStruct((nchunks, CHUNK), jnp.float32)),
    )(edge_index, edge_weight)


def _dense_body(ego_ref, lp_ref, w1_ref, w2_ref, b1_ref, b2_ref, out_ref):
    ego = ego_ref[...]
    lap = lp_ref[0] + lp_ref[1]
    y = jnp.dot(ego + lap, w1_ref[...], preferred_element_type=jnp.float32)
    y += jnp.dot(ego * lap, w2_ref[...], preferred_element_type=jnp.float32)
    y += b1_ref[...] + b2_ref[...]
    y = jnp.where(y >= 0, y, 0.2 * y)
    norm = jnp.sqrt(jnp.sum(y * y, axis=1, keepdims=True))
    out_ref[...] = y / jnp.maximum(norm, 1e-12)


def _dense(ego, partials, W1, W2, b1, b2):
    R = 2000
    grid = (N_NODES // R,)
    row_spec = pl.BlockSpec((R, D), lambda i: (i, 0))
    part_spec = pl.BlockSpec((NUM_SC, R, D), lambda i: (0, i, 0))
    full_spec = pl.BlockSpec((D, D), lambda i: (0, 0))
    bias_spec = pl.BlockSpec((1, D), lambda i: (0, 0))
    return pl.pallas_call(
        _dense_body,
        grid=grid,
        in_specs=[row_spec, part_spec, full_spec, full_spec,
                  bias_spec, bias_spec],
        out_specs=row_spec,
        out_shape=jax.ShapeDtypeStruct((N_NODES, D), jnp.float32),
    )(ego, partials, W1, W2, b1, b2)


@jax.jit
def kernel(ego_embeddings, edge_index, edge_weight, W1, W2, b1, b2):
    ei = edge_index.astype(jnp.int32).reshape(2, _E_ROWS, CHUNK)
    wr = edge_weight.astype(jnp.float32).reshape(_E_ROWS, CHUNK)
    packed, w = _prep(ei, wr)
    partials = _sc_segment_sum(ego_embeddings, packed, w)
    return _dense(ego_embeddings, partials, W1, W2, b1, b2)
